# Initial kernel scaffold; baseline (speedup 1.0000x reference)
import jax
import jax.numpy as jnp
from jax.experimental import pallas as pl


def kernel(chunk_scores):
    raise NotImplementedError("write your pallas kernel here")


# speedup vs baseline: 10.3807x; 10.3807x over previous
import functools
import jax
import jax.numpy as jnp
from jax.experimental import pallas as pl
from jax.experimental.pallas import tpu as pltpu

_K = 64


def _tc_body(x_ref, w_ref, fs_ref, bi_ref):
    x = x_ref[...]
    B, N = x.shape
    k = _K

    int_min = jnp.int32(-2147483648)
    mask31 = jnp.int32(0x7FFFFFFF)
    b = jax.lax.bitcast_convert_type(x, jnp.int32)
    key = jnp.where(b >= 0, b, b ^ mask31)
    key = jnp.where(b == int_min, jnp.int32(0), key)

    iota = jax.lax.broadcasted_iota(jnp.int32, (B, N), 1)

    rowmax = jnp.max(key, axis=1, keepdims=True)
    bidx = jnp.min(jnp.where(key == rowmax, iota, jnp.int32(N)), axis=1)

    is_nonneg = key >= 0
    c_pos = jnp.sum(is_nonneg, axis=1, keepdims=True, dtype=jnp.int32)
    pos_class = c_pos >= k
    valid = is_nonneg == pos_class
    mag = key & mask31
    kk = jnp.where(pos_class, jnp.int32(k), k - c_pos)

    p = jnp.zeros((B, 1), jnp.int32)
    for bit in range(30, -1, -1):
        cand = p | jnp.int32(1 << bit)
        c = jnp.sum(valid & (mag >= cand), axis=1, keepdims=True,
                    dtype=jnp.int32)
        p = jnp.where(c >= kk, cand, p)
    t_key = jnp.where(pos_class, p, p | int_min)

    gt = key > t_key
    eq = key == t_key
    c_gt = jnp.sum(gt, axis=1, keepdims=True, dtype=jnp.int32)
    r = k - c_gt

    q = jnp.zeros((B, 1), jnp.int32)
    for bit in range(13, -1, -1):
        cand = q | jnp.int32(1 << bit)
        c = jnp.sum(eq & (iota < cand), axis=1, keepdims=True,
                    dtype=jnp.int32)
        q = jnp.where(c < r, cand, q)
    sel = gt | (eq & (iota <= q))

    w_ref[...] = jnp.where(sel, jnp.float32(1.0 / k), jnp.float32(0.0))

    t_bits = jnp.where(t_key >= 0, t_key, t_key ^ mask31)
    t_val = jax.lax.bitcast_convert_type(t_bits, jnp.float32)
    sum_gt = jnp.sum(jnp.where(gt, x, jnp.float32(0.0)), axis=1,
                     keepdims=True)
    fs = (sum_gt + r.astype(jnp.float32) * t_val) * jnp.float32(1.0 / k)

    fs_ref[...] = jnp.broadcast_to(fs, fs_ref.shape)
    bi_ref[...] = jnp.broadcast_to(bidx[:, None], bi_ref.shape)


@jax.jit
def kernel(chunk_scores):
    B, N = chunk_scores.shape
    w, fs, bi = pl.pallas_call(
        _tc_body,
        out_shape=(
            jax.ShapeDtypeStruct((B, N), jnp.float32),
            jax.ShapeDtypeStruct((B, 128), jnp.float32),
            jax.ShapeDtypeStruct((B, 128), jnp.int32),
        ),
    )(chunk_scores)
    return fs[:, 0], bi[:, 0], w
